# Initial kernel scaffold; baseline (speedup 1.0000x reference)
import jax
import jax.numpy as jnp
from jax.experimental import pallas as pl


def kernel(x, edge_index, W1, b1, W2, b2):
    raise NotImplementedError("write your pallas kernel here")


# speedup vs baseline: 19.6969x; 19.6969x over previous
import functools

import jax
import jax.numpy as jnp
from jax import lax
from jax.experimental import pallas as pl
from jax.experimental.pallas import tpu as pltpu
from jax.experimental.pallas import tpu_sc as plsc

N = 10000
NP = 10240
D = 128
E = 320000
NC = 2
NS = 16
NW = NC * NS
EPW = E // NW
CHUNK = 80
NCHUNK = EPW // CHUNK
RPT = NP // NS
BLK = 1024
NBLK = NP // BLK

_mesh = plsc.VectorSubcoreMesh(core_axis_name="c", subcore_axis_name="s")


@functools.partial(
    pl.kernel,
    out_type=jax.ShapeDtypeStruct((NC, NP), jnp.float32),
    mesh=_mesh,
    scratch_types=[
        pltpu.VMEM((CHUNK,), jnp.float32),
        pltpu.VMEM((NCHUNK, CHUNK), jnp.int32),
        pltpu.VMEM_SHARED((NP,), jnp.float32),
    ],
)
def _deg_kernel(dst_hbm, zeros_hbm, degp_hbm, ones_v, idx_v, deg_sh):
    c = lax.axis_index("c")
    s = lax.axis_index("s")
    wid = s * NC + c
    for i in range(CHUNK // 16):
        ones_v[pl.ds(i * 16, 16)] = jnp.full((16,), 1.0, jnp.float32)

    @pl.when(s == 0)
    def _():
        pltpu.sync_copy(zeros_hbm, deg_sh)

    pltpu.sync_copy(dst_hbm.at[wid], idx_v)
    plsc.subcore_barrier()

    def step(j, carry):
        pltpu.sync_copy(ones_v, deg_sh.at[idx_v.at[j]], add=True)
        return carry

    lax.fori_loop(0, NCHUNK, step, 0)
    plsc.subcore_barrier()

    @pl.when(s == 0)
    def _():
        pltpu.sync_copy(deg_sh, degp_hbm.at[c])


@functools.partial(
    pl.kernel,
    out_type=jax.ShapeDtypeStruct((NC, NP, D), jnp.float32),
    mesh=_mesh,
    scratch_types=[
        pltpu.VMEM((NCHUNK, CHUNK), jnp.int32),
        pltpu.VMEM((NCHUNK, CHUNK), jnp.int32),
        pltpu.VMEM((CHUNK, D), jnp.float32),
        pltpu.VMEM_SHARED((NP, D), jnp.float32),
        pltpu.SemaphoreType.DMA,
    ],
)
def _msg_kernel(g_hbm, src_hbm, dst_hbm, zeros_hbm, accp_hbm,
                idxs_v, idxd_v, rows_v, acc_sh, gsem):
    c = lax.axis_index("c")
    s = lax.axis_index("s")
    wid = s * NC + c
    pltpu.sync_copy(zeros_hbm, acc_sh.at[pl.ds(s * RPT, RPT)])
    pltpu.sync_copy(src_hbm.at[wid], idxs_v)
    pltpu.sync_copy(dst_hbm.at[wid], idxd_v)
    plsc.subcore_barrier()

    def step(j, carry):
        pltpu.async_copy(g_hbm.at[idxs_v.at[j]], rows_v, gsem).wait()
        pltpu.sync_copy(rows_v, acc_sh.at[idxd_v.at[j]], add=True)
        return carry

    lax.fori_loop(0, NCHUNK, step, 0)
    plsc.subcore_barrier()
    pltpu.sync_copy(acc_sh.at[pl.ds(s * RPT, RPT)],
                    accp_hbm.at[c, pl.ds(s * RPT, RPT)])


def _mm_body(x_ref, w_ref, o_ref):
    o_ref[...] = jnp.dot(x_ref[...], w_ref[...],
                         preferred_element_type=jnp.float32)


def _matmul(xp, W):
    return pl.pallas_call(
        _mm_body,
        grid=(NBLK,),
        in_specs=[pl.BlockSpec((BLK, D), lambda i: (i, 0)),
                  pl.BlockSpec((D, D), lambda i: (0, 0))],
        out_specs=pl.BlockSpec((BLK, D), lambda i: (i, 0)),
        out_shape=jax.ShapeDtypeStruct((NP, D), jnp.float32),
    )(xp, W)


def _scale_body(degp_ref, h_ref, dis_ref, g_ref):
    deg = degp_ref[0, :] + degp_ref[1, :] + 1.0
    dis = lax.rsqrt(deg)
    dis_ref[...] = dis
    g_ref[...] = h_ref[...] * dis[:, None]


def _scale(degp, h):
    return pl.pallas_call(
        _scale_body,
        grid=(NBLK,),
        in_specs=[pl.BlockSpec((NC, BLK), lambda i: (0, i)),
                  pl.BlockSpec((BLK, D), lambda i: (i, 0))],
        out_specs=[pl.BlockSpec((BLK,), lambda i: (i,)),
                   pl.BlockSpec((BLK, D), lambda i: (i, 0))],
        out_shape=[jax.ShapeDtypeStruct((NP,), jnp.float32),
                   jax.ShapeDtypeStruct((NP, D), jnp.float32)],
    )(degp, h)


def _elu(z):
    return jnp.where(z > 0.0, z, jnp.exp(jnp.minimum(z, 0.0)) - 1.0)


def _layer_body(acc_ref, g_ref, dis_ref, b_ref, w_ref, g2_ref):
    dis = dis_ref[...]
    t = acc_ref[0] + acc_ref[1] + g_ref[...]
    z = _elu(t * dis[:, None] + b_ref[...][None, :])
    h2 = jnp.dot(z, w_ref[...], preferred_element_type=jnp.float32)
    g2_ref[...] = h2 * dis[:, None]


def _layer(accp, g, dis, b, W):
    return pl.pallas_call(
        _layer_body,
        grid=(NBLK,),
        in_specs=[pl.BlockSpec((NC, BLK, D), lambda i: (0, i, 0)),
                  pl.BlockSpec((BLK, D), lambda i: (i, 0)),
                  pl.BlockSpec((BLK,), lambda i: (i,)),
                  pl.BlockSpec((D,), lambda i: (0,)),
                  pl.BlockSpec((D, D), lambda i: (0, 0))],
        out_specs=pl.BlockSpec((BLK, D), lambda i: (i, 0)),
        out_shape=jax.ShapeDtypeStruct((NP, D), jnp.float32),
    )(accp, g, dis, b, W)


def _final_body(acc_ref, g_ref, dis_ref, b_ref, o_ref):
    dis = dis_ref[...]
    t = acc_ref[0] + acc_ref[1] + g_ref[...]
    o_ref[...] = _elu(t * dis[:, None] + b_ref[...][None, :])


def _final(accp, g, dis, b):
    return pl.pallas_call(
        _final_body,
        grid=(NBLK,),
        in_specs=[pl.BlockSpec((NC, BLK, D), lambda i: (0, i, 0)),
                  pl.BlockSpec((BLK, D), lambda i: (i, 0)),
                  pl.BlockSpec((BLK,), lambda i: (i,)),
                  pl.BlockSpec((D,), lambda i: (0,))],
        out_specs=pl.BlockSpec((BLK, D), lambda i: (i, 0)),
        out_shape=jax.ShapeDtypeStruct((NP, D), jnp.float32),
    )(accp, g, dis, b)


def kernel(x, edge_index, W1, b1, W2, b2):
    ei = edge_index.astype(jnp.int32)
    src = ei[0].reshape(NW, NCHUNK, CHUNK)
    dst = ei[1].reshape(NW, NCHUNK, CHUNK)
    xp = jnp.pad(x, ((0, NP - N), (0, 0)))
    zeros1 = jnp.zeros((NP,), jnp.float32)
    zeros2 = jnp.zeros((RPT, D), jnp.float32)

    degp = _deg_kernel(dst, zeros1)
    h1 = _matmul(xp, W1)
    dis, g1 = _scale(degp, h1)
    acc1 = _msg_kernel(g1, src, dst, zeros2)
    g2 = _layer(acc1, g1, dis, b1, W2)
    acc2 = _msg_kernel(g2, src, dst, zeros2)
    outp = _final(acc2, g2, dis, b2)
    return outp[:N]
